# baseline (speedup 1.0000x reference)
import functools

import jax
import jax.numpy as jnp
from jax import lax
from jax.experimental import pallas as pl
from jax.experimental.pallas import tpu as pltpu
from jax.experimental.pallas import tpu_sc as plsc

HD, ID, NE, TOPK = 768, 3072, 8, 2
TOK = 2048
ASSIGN = TOK * TOPK
AUX = 0.001
TILE = 256
NT = 24
NP = NT * TILE
IC = 1536
NIC = ID // IC
NC, NS = 2, 16
NW = NC * NS
SPW = NP // NW
GCH = 64
TPW = TOK // NW


def _router_kernel(x_ref, wr_ref, p0_ref, p1_ref, rw0_ref, rw1_ref,
                   te_ref, ntu_ref, lb_ref, z_ref):
    x = x_ref[...]
    logits = jnp.dot(x, wr_ref[...], preferred_element_type=jnp.float32)
    m = jnp.max(logits, axis=-1, keepdims=True)
    ex = jnp.exp(logits - m)
    se = jnp.sum(ex, axis=-1, keepdims=True)
    probs = ex / se
    eidx = lax.broadcasted_iota(jnp.int32, probs.shape, 1)
    v1 = jnp.max(probs, axis=-1, keepdims=True)
    i1 = jnp.min(jnp.where(probs == v1, eidx, NE), axis=-1, keepdims=True)
    oh1 = eidx == i1
    probs_m = jnp.where(oh1, -1.0, probs)
    v2 = jnp.max(probs_m, axis=-1, keepdims=True)
    i2 = jnp.min(jnp.where(probs_m == v2, eidx, NE), axis=-1, keepdims=True)
    oh2 = eidx == i2
    denom = v1 + v2
    rw0_ref[...] = v1 / denom
    rw1_ref[...] = v2 / denom
    oh1f = oh1.astype(jnp.float32)
    oh2f = oh2.astype(jnp.float32)
    cnt1 = jnp.sum(oh1f, axis=0, keepdims=True)
    cnt2 = jnp.sum(oh2f, axis=0, keepdims=True)
    counts = cnt1 + cnt2
    mean_probs = jnp.mean(probs, axis=0, keepdims=True)
    lb_ref[...] = (AUX * NE * jnp.sum(counts / (TOK * TOPK) * mean_probs)
                   ).reshape(1, 1)
    lse = m + jnp.log(se)
    z_ref[...] = jnp.mean(lse * lse).reshape(1, 1)
    ohb = jnp.concatenate([oh1f, oh2f], axis=1)
    ti = lax.broadcasted_iota(jnp.int32, (TOK, 1), 0)
    tj = lax.broadcasted_iota(jnp.int32, (1, TOK), 1)
    tri = (tj < ti).astype(jnp.bfloat16)
    pref = jnp.dot(tri, ohb.astype(jnp.bfloat16),
                   preferred_element_type=jnp.float32)
    p1x = pref[:, :NE]
    p2x = pref[:, NE:]
    pc = jnp.ceil(counts / TILE) * TILE
    ei = lax.broadcasted_iota(jnp.int32, (NE, NE), 0)
    ej = lax.broadcasted_iota(jnp.int32, (NE, NE), 1)
    ustrict = (ei < ej).astype(jnp.bfloat16)
    start = jnp.dot(pc.astype(jnp.bfloat16), ustrict,
                    preferred_element_type=jnp.float32)
    ntu_ref[...] = (jnp.sum(pc) / TILE).astype(jnp.int32).reshape(1, 1)
    rank0 = jnp.sum(jnp.where(oh1, p1x, 0.0), axis=1, keepdims=True)
    st0 = jnp.sum(start * oh1f, axis=1, keepdims=True)
    p0_ref[...] = (rank0 + st0).astype(jnp.int32)
    rank1 = jnp.sum(jnp.where(oh2, p2x, 0.0), axis=1, keepdims=True)
    base1 = jnp.sum(cnt1 * oh2f, axis=1, keepdims=True)
    st1 = jnp.sum(start * oh2f, axis=1, keepdims=True)
    p1_ref[...] = (rank1 + base1 + st1).astype(jnp.int32)
    jt = lax.broadcasted_iota(jnp.int32, (128, NE), 0) * TILE
    te = jnp.sum((jnp.broadcast_to(start, (128, NE)) <= jt.astype(jnp.float32)
                  ).astype(jnp.int32), axis=1, keepdims=True) - 1
    te_ref[...] = te


def _dispatch_kernel(p0_hbm, p1_hbm, rw0_hbm, rw1_hbm, x_hbm,
                     xs_hbm, wgt_hbm,
                     tok_v, wgt_v, pk_v, rwk_v, idx_v, rows_v, tok_sh, sem):
    cid = lax.axis_index("c")
    sid = lax.axis_index("s")
    wid = sid * NC + cid

    @pl.when(sid == 0)
    def _scatter_phase():
        zi = jnp.zeros((16,), jnp.int32)
        zf = jnp.zeros((16,), jnp.float32)

        def _zero(i, _):
            tok_v[pl.ds(i * 16, 16)] = zi
            wgt_v[pl.ds(i * 16, 16)] = zf
            return _
        lax.fori_loop(0, NP // 16, _zero, None)
        pltpu.sync_copy(p0_hbm, pk_v.at[pl.ds(0, TOK)])
        pltpu.sync_copy(p1_hbm, pk_v.at[pl.ds(TOK, TOK)])
        pltpu.sync_copy(rw0_hbm, rwk_v.at[pl.ds(0, TOK)])
        pltpu.sync_copy(rw1_hbm, rwk_v.at[pl.ds(TOK, TOK)])
        lanes = lax.broadcasted_iota(jnp.int32, (16,), 0)

        def _scat(c, _):
            pv = pk_v[pl.ds(c * 16, 16)]
            wv = rwk_v[pl.ds(c * 16, 16)]
            tv = (c * 16 % TOK) + lanes
            plsc.store_scatter(tok_v, [pv], tv)
            plsc.store_scatter(wgt_v, [pv], wv)
            return _
        lax.fori_loop(0, ASSIGN // 16, _scat, None)
        pltpu.sync_copy(tok_v, tok_sh)

        @pl.when(cid == 0)
        def _():
            pltpu.sync_copy(wgt_v, wgt_hbm)

    plsc.subcore_barrier()
    for c in range(SPW // GCH):
        base = wid * SPW + c * GCH
        pltpu.sync_copy(tok_sh.at[pl.ds(base, GCH)], idx_v)
        pltpu.async_copy(x_hbm.at[idx_v], rows_v, sem).wait()
        pltpu.sync_copy(rows_v, xs_hbm.at[pl.ds(base, GCH)])


def _ffn_kernel(te_ref, ntu_ref, xs_ref, wgt_ref, w1_ref, w2_ref, ys_ref):
    j = pl.program_id(0)
    ic = pl.program_id(1)

    @pl.when(j < ntu_ref[0])
    def _():
        xb = xs_ref[...].astype(jnp.bfloat16)
        h = jnp.dot(xb, w1_ref[0].astype(jnp.bfloat16),
                    preferred_element_type=jnp.float32)
        h = jax.nn.gelu(h)
        part = jnp.dot(h.astype(jnp.bfloat16), w2_ref[0].astype(jnp.bfloat16),
                       preferred_element_type=jnp.float32)
        contrib = wgt_ref[...] * part

        @pl.when(ic == 0)
        def _():
            ys_ref[...] = contrib

        @pl.when(ic > 0)
        def _():
            ys_ref[...] += contrib


def _combine_kernel(p0_hbm, p1_hbm, ys_hbm, g0_hbm, g1_hbm,
                    idx_v, rows_v, sem):
    cid = lax.axis_index("c")
    sid = lax.axis_index("s")
    wid = sid * NC + cid
    base = wid * TPW
    pltpu.sync_copy(p0_hbm.at[pl.ds(base, TPW)], idx_v)
    pltpu.async_copy(ys_hbm.at[idx_v], rows_v, sem).wait()
    pltpu.sync_copy(rows_v, g0_hbm.at[pl.ds(base, TPW)])
    pltpu.sync_copy(p1_hbm.at[pl.ds(base, TPW)], idx_v)
    pltpu.async_copy(ys_hbm.at[idx_v], rows_v, sem).wait()
    pltpu.sync_copy(rows_v, g1_hbm.at[pl.ds(base, TPW)])


def _add_kernel(a_ref, b_ref, o_ref):
    o_ref[...] = a_ref[...] + b_ref[...]


def kernel(hidden_states, Wr, w1, w2):
    b, s, h = hidden_states.shape
    x = hidden_states.reshape(-1, h).astype(jnp.float32)

    p0, p1, rw0, rw1, te, ntu, lb, z = pl.pallas_call(
        _router_kernel,
        out_shape=[
            jax.ShapeDtypeStruct((TOK, 1), jnp.int32),
            jax.ShapeDtypeStruct((TOK, 1), jnp.int32),
            jax.ShapeDtypeStruct((TOK, 1), jnp.float32),
            jax.ShapeDtypeStruct((TOK, 1), jnp.float32),
            jax.ShapeDtypeStruct((128, 1), jnp.int32),
            jax.ShapeDtypeStruct((1, 1), jnp.int32),
            jax.ShapeDtypeStruct((1, 1), jnp.float32),
            jax.ShapeDtypeStruct((1, 1), jnp.float32),
        ],
    )(x, Wr)

    p0f = p0.reshape(TOK)
    p1f = p1.reshape(TOK)
    te_arr = te.reshape(128)[:NT]
    ntu_arr = ntu.reshape(1)

    mesh = plsc.VectorSubcoreMesh(core_axis_name="c", subcore_axis_name="s")
    xs, wgt = pl.kernel(
        _dispatch_kernel,
        out_type=[
            jax.ShapeDtypeStruct((NP, HD), jnp.float32),
            jax.ShapeDtypeStruct((NP,), jnp.float32),
        ],
        mesh=mesh,
        compiler_params=pltpu.CompilerParams(needs_layout_passes=False),
        scratch_types=[
            pltpu.VMEM((NP,), jnp.int32),
            pltpu.VMEM((NP,), jnp.float32),
            pltpu.VMEM((ASSIGN,), jnp.int32),
            pltpu.VMEM((ASSIGN,), jnp.float32),
            pltpu.VMEM((GCH,), jnp.int32),
            pltpu.VMEM((GCH, HD), jnp.float32),
            pltpu.VMEM_SHARED((NP,), jnp.int32),
            pltpu.SemaphoreType.DMA,
        ],
    )(p0f, p1f, rw0.reshape(TOK), rw1.reshape(TOK), x)

    grid_spec = pltpu.PrefetchScalarGridSpec(
        num_scalar_prefetch=2,
        grid=(NT, NIC),
        in_specs=[
            pl.BlockSpec((TILE, HD), lambda j, ic, te, ntu: (j, 0)),
            pl.BlockSpec((TILE, 1), lambda j, ic, te, ntu: (j, 0)),
            pl.BlockSpec((1, HD, IC), lambda j, ic, te, ntu: (te[j], 0, ic)),
            pl.BlockSpec((1, IC, HD), lambda j, ic, te, ntu: (te[j], ic, 0)),
        ],
        out_specs=pl.BlockSpec((TILE, HD), lambda j, ic, te, ntu: (j, 0)),
    )
    ys = pl.pallas_call(
        _ffn_kernel,
        grid_spec=grid_spec,
        out_shape=jax.ShapeDtypeStruct((NP, HD), jnp.float32),
    )(te_arr, ntu_arr, xs, wgt.reshape(NP, 1), w1, w2)

    g0, g1 = pl.kernel(
        _combine_kernel,
        out_type=[
            jax.ShapeDtypeStruct((TOK, HD), jnp.float32),
            jax.ShapeDtypeStruct((TOK, HD), jnp.float32),
        ],
        mesh=mesh,
        compiler_params=pltpu.CompilerParams(needs_layout_passes=False),
        scratch_types=[
            pltpu.VMEM((TPW,), jnp.int32),
            pltpu.VMEM((TPW, HD), jnp.float32),
            pltpu.SemaphoreType.DMA,
        ],
    )(p0f, p1f, ys)

    out = pl.pallas_call(
        _add_kernel,
        out_shape=jax.ShapeDtypeStruct((TOK, HD), jnp.float32),
    )(g0, g1)

    return out.reshape(b, s, h), lb[0, 0], z[0, 0]
